# baseline (speedup 1.0000x reference)
import functools

import jax
import jax.numpy as jnp
from jax import lax
from jax.experimental import pallas as pl
from jax.experimental.pallas import tpu as pltpu
from jax.experimental.pallas import tpu_sc as plsc

_LANES = 16
_EPS = 1e-5


@functools.lru_cache(maxsize=None)
def _build_transpose(V, H):
    info = plsc.get_sparse_core_info()
    NC, NS = info.num_cores, info.num_subcores
    NW = NC * NS
    assert H == 4 * _LANES and V % 128 == 64
    NFULL = V // 128
    PER = -(-NFULL // NW)
    CW = 129

    mesh = plsc.VectorSubcoreMesh(core_axis_name="c", subcore_axis_name="s")

    @functools.partial(
        pl.kernel,
        mesh=mesh,
        out_type=jax.ShapeDtypeStruct((V // 2, 2 * H), jnp.float32),
        compiler_params=pltpu.CompilerParams(
            needs_layout_passes=False, use_tc_tiling_on_sc=True),
        scratch_types=[
            pltpu.VMEM((2, H, 128), jnp.float32),
            pltpu.VMEM((2, H, CW), jnp.float32),
            pltpu.SemaphoreType.DMA,
            pltpu.SemaphoreType.DMA,
            pltpu.SemaphoreType.DMA,
            pltpu.SemaphoreType.DMA,
        ],
    )
    def tbody(tT_hbm, tail2_hbm, t2_hbm, src_v, dst_v, i0, i1, o0, o1):
        wid = lax.axis_index("s") * NC + lax.axis_index("c")
        c_lo = wid * PER
        c_hi = jnp.minimum(c_lo + PER, NFULL)
        iota = lax.iota(jnp.int32, _LANES)
        pvec = [(jnp.int32(16 * gg) + iota) // 2 for gg in range(8)]
        qbase = [((jnp.int32(16 * gg) + iota) % 2) * H for gg in range(8)]

        def fire_in(c, b, sem):
            pltpu.async_copy(
                tT_hbm.at[:, pl.ds(c * 128, 128)], src_v.at[b], sem)

        def wait_in(b, sem):
            pltpu.make_async_copy(
                tT_hbm.at[:, pl.ds(0, 128)], src_v.at[b], sem).wait()

        def fire_out(c, b, sem):
            pltpu.async_copy(
                dst_v.at[b, pl.ds(0, H), pl.ds(0, 2 * H)],
                t2_hbm.at[pl.ds(c * H, H)], sem)

        def wait_out(b, sem):
            pltpu.make_async_copy(
                dst_v.at[b, pl.ds(0, H), pl.ds(0, 2 * H)],
                t2_hbm.at[pl.ds(0, H)], sem).wait()

        def transpose_block(b, width):
            ngg = width // _LANES

            @plsc.parallel_loop(0, H, 2, unroll=2)
            def _h2(h):
                for hk in range(2):
                    for gg in range(ngg):
                        plsc.store_scatter(
                            dst_v.at[b],
                            [pvec[gg], qbase[gg] + (h + hk)],
                            src_v[b, h + hk, pl.ds(16 * gg, _LANES)],
                        )

        @pl.when(c_lo < c_hi)
        def _():
            fire_in(c_lo, 0, i0)

        def body2(k2, carry):
            c0 = c_lo + 2 * k2
            c1 = c0 + 1

            @pl.when(c1 < c_hi)
            def _():
                fire_in(c1, 1, i1)

            @pl.when(c0 < c_hi)
            def _():
                wait_in(0, i0)

                @pl.when(k2 > 0)
                def _():
                    wait_out(0, o0)

                transpose_block(0, 128)
                fire_out(c0, 0, o0)

            @pl.when(c0 + 2 < c_hi)
            def _():
                fire_in(c0 + 2, 0, i0)

            @pl.when(c1 < c_hi)
            def _():
                wait_in(1, i1)

                @pl.when(k2 > 0)
                def _():
                    wait_out(1, o1)

                transpose_block(1, 128)
                fire_out(c1, 1, o1)

            return carry

        lax.fori_loop(0, (PER + 1) // 2, body2, 0)

        @pl.when(c_lo < c_hi)
        def _():
            wait_out(0, o0)

        @pl.when(c_lo + 1 < c_hi)
        def _():
            wait_out(1, o1)

        @pl.when(wid == NW - 1)
        def _():
            pltpu.sync_copy(
                tail2_hbm, src_v.at[0, pl.ds(0, 32), pl.ds(0, 2 * H)])
            pltpu.sync_copy(
                src_v.at[0, pl.ds(0, 32), pl.ds(0, 2 * H)],
                t2_hbm.at[pl.ds(NFULL * H, 32)])

    return tbody


@functools.lru_cache(maxsize=None)
def _build(B, L, V, H):
    info = plsc.get_sparse_core_info()
    NC, NS = info.num_cores, info.num_subcores
    NW = NC * NS
    assert B % (NW * 128) == 0 and H == 4 * _LANES and L % 2 == 0
    HJ = H // _LANES
    CW = 129

    mesh = plsc.VectorSubcoreMesh(core_axis_name="c", subcore_axis_name="s")

    @functools.partial(
        pl.kernel,
        mesh=mesh,
        out_type=jax.ShapeDtypeStruct((L, H // 8, 32, 8, 128), jnp.float32),
        compiler_params=pltpu.CompilerParams(
            needs_layout_passes=False, use_tc_tiling_on_sc=False),
        scratch_types=[
            pltpu.VMEM((2, 128), jnp.int32),
            pltpu.VMEM((2, 128, H), jnp.float32),
            pltpu.VMEM((2, H, CW), jnp.float32),
            pltpu.VMEM((L, H), jnp.float32),
            pltpu.VMEM((H,), jnp.float32),
            pltpu.VMEM((H,), jnp.float32),
            pltpu.SemaphoreType.DMA,
            pltpu.SemaphoreType.DMA,
            pltpu.SemaphoreType.DMA,
            pltpu.SemaphoreType.DMA,
        ],
    )
    def body(tokt_hbm, table_hbm, pos_hbm, gamma_hbm, beta_hbm, out_hbm,
             idx_v, rows_v, outs_v, pos_v, gamma_v, beta_v, g0, g1, o0, o1):
        wid = lax.axis_index("s") * NC + lax.axis_index("c")
        b0 = wid * 128
        pltpu.sync_copy(pos_hbm, pos_v)
        pltpu.sync_copy(gamma_hbm, gamma_v)
        pltpu.sync_copy(beta_hbm, beta_v)
        g = [gamma_v[pl.ds(j * _LANES, _LANES)] for j in range(HJ)]
        bta = [beta_v[pl.ds(j * _LANES, _LANES)] for j in range(HJ)]
        hvec = [jnp.int32(j * _LANES)
                + lax.iota(jnp.int32, _LANES) for j in range(HJ)]

        def fire_gather(l, b, gsem):
            pltpu.sync_copy(tokt_hbm.at[l, pl.ds(b0, 128)], idx_v.at[b])
            pltpu.async_copy(table_hbm.at[idx_v.at[b]], rows_v.at[b], gsem)

        def wait_gather(b, gsem):
            pltpu.make_async_copy(
                table_hbm.at[idx_v.at[b]], rows_v.at[b], gsem).wait()

        def fire_out(l, b, osem):
            for ht in range(H // 8):
                pltpu.async_copy(
                    outs_v.at[b, pl.ds(ht * 8, 8), pl.ds(0, 128)],
                    out_hbm.at[l, ht, wid],
                    osem,
                )

        def wait_out(b, osem):
            for ht in range(H // 8):
                pltpu.make_async_copy(
                    outs_v.at[b, pl.ds(ht * 8, 8), pl.ds(0, 128)],
                    out_hbm.at[0, ht, wid],
                    osem,
                ).wait()

        def compute(l, b):
            p = [pos_v[l, pl.ds(j * _LANES, _LANES)] for j in range(HJ)]
            KU = 2
            magic = jnp.full((_LANES,), 0x5F3759DF, dtype=jnp.int32)

            @plsc.parallel_loop(0, 128, KU, unroll=2)
            def row_group(i4):
                rr = [i4 + k for k in range(KU)]
                y = [[rows_v[b, r, pl.ds(j * _LANES, _LANES)] + p[j]
                      for j in range(HJ)] for r in rr]
                s = [(yk[0] + yk[1]) + (yk[2] + yk[3]) for yk in y]
                t = [(yk[0] * yk[0] + yk[1] * yk[1])
                     + (yk[2] * yk[2] + yk[3] * yk[3]) for yk in y]
                sr = [lax.rev(sk, (0,)) for sk in s]
                tr = [lax.rev(tk, (0,)) for tk in t]
                cs = [jnp.cumsum(sk) for sk in s]
                csr = [jnp.cumsum(sk) for sk in sr]
                ct = [jnp.cumsum(tk) for tk in t]
                ctr = [jnp.cumsum(tk) for tk in tr]
                tot_s = [c + lax.rev(cr, (0,)) - sk
                         for c, cr, sk in zip(cs, csr, s)]
                tot_t = [c + lax.rev(cr, (0,)) - tk
                         for c, cr, tk in zip(ct, ctr, t)]
                mean = [v * (1.0 / H) for v in tot_s]
                var = [v * (1.0 / H) - m * m
                       for v, m in zip(tot_t, mean)]
                a = [v + _EPS for v in var]
                rs = [plsc.bitcast(
                    magic - lax.shift_right_arithmetic(
                        plsc.bitcast(ak, jnp.int32), 1),
                    jnp.float32) for ak in a]
                rs = [rk * (1.5 - 0.5 * ak * rk * rk)
                      for rk, ak in zip(rs, a)]
                rvec = [jnp.full((_LANES,), r, dtype=jnp.int32) for r in rr]
                for k in range(KU):
                    for j in range(HJ):
                        plsc.store_scatter(
                            outs_v.at[b],
                            [hvec[j], rvec[k]],
                            (y[k][j] - mean[k]) * (rs[k] * g[j]) + bta[j],
                        )

        fire_gather(0, 0, g0)

        def body2(l2, carry):
            l0 = 2 * l2
            l1 = l0 + 1
            fire_gather(l1, 1, g1)
            wait_gather(0, g0)

            @pl.when(l2 > 0)
            def _():
                wait_out(0, o0)

            compute(l0, 0)
            fire_out(l0, 0, o0)

            @pl.when(l2 < L // 2 - 1)
            def _():
                fire_gather(l0 + 2, 0, g0)

            wait_gather(1, g1)

            @pl.when(l2 > 0)
            def _():
                wait_out(1, o1)

            compute(l1, 1)
            fire_out(l1, 1, o1)
            return carry

        lax.fori_loop(0, L // 2, body2, 0)
        wait_out(0, o0)
        wait_out(1, o1)

    return body


def kernel(tokens, start_pos, token_table, pos_table, gamma, beta):
    B, L = tokens.shape
    V, H = token_table.shape
    tokt = tokens.T.astype(jnp.int32)
    pos_slice = lax.dynamic_slice_in_dim(pos_table, start_pos, L, axis=0)
    tbody = _build_transpose(V, H)
    tail2 = token_table[V - V % 128:].reshape(-1, 2 * H)
    table_lin = tbody(token_table.T, tail2).reshape(V, H)
    body = _build(B, L, V, H)
    out5 = body(tokt, table_lin, pos_slice, gamma, beta)
    return out5.transpose(2, 4, 0, 1, 3).reshape(B, L, H)


# speedup vs baseline: 1.2332x; 1.2332x over previous
import functools

import jax
import jax.numpy as jnp
from jax import lax
from jax.experimental import pallas as pl
from jax.experimental.pallas import tpu as pltpu
from jax.experimental.pallas import tpu_sc as plsc

_LANES = 16
_EPS = 1e-5


@functools.lru_cache(maxsize=None)
def _build(B, L, V, H):
    info = plsc.get_sparse_core_info()
    NC, NS = info.num_cores, info.num_subcores
    NW = NC * NS
    assert B % (NW * 128) == 0 and H == 4 * _LANES and L % 2 == 0
    HJ = H // _LANES
    CW = 129

    mesh = plsc.VectorSubcoreMesh(core_axis_name="c", subcore_axis_name="s")

    @functools.partial(
        pl.kernel,
        mesh=mesh,
        out_type=jax.ShapeDtypeStruct((L, H // 8, 32, 8, 128), jnp.float32),
        compiler_params=pltpu.CompilerParams(
            needs_layout_passes=False, use_tc_tiling_on_sc=False),
        scratch_types=[
            pltpu.VMEM((2, 128), jnp.int32),
            pltpu.VMEM((2, 128, H), jnp.float32),
            pltpu.VMEM((2, H, CW), jnp.float32),
            pltpu.VMEM((L, H), jnp.float32),
            pltpu.VMEM((H,), jnp.float32),
            pltpu.VMEM((H,), jnp.float32),
            pltpu.SemaphoreType.DMA,
            pltpu.SemaphoreType.DMA,
            pltpu.SemaphoreType.DMA,
            pltpu.SemaphoreType.DMA,
        ],
    )
    def body(tokt_hbm, table_hbm, pos_hbm, gamma_hbm, beta_hbm, out_hbm,
             idx_v, rows_v, outs_v, pos_v, gamma_v, beta_v, g0, g1, o0, o1):
        wid = lax.axis_index("s") * NC + lax.axis_index("c")
        b0 = wid * 128
        pltpu.sync_copy(pos_hbm, pos_v)
        pltpu.sync_copy(gamma_hbm, gamma_v)
        pltpu.sync_copy(beta_hbm, beta_v)
        g = [gamma_v[pl.ds(j * _LANES, _LANES)] for j in range(HJ)]
        bta = [beta_v[pl.ds(j * _LANES, _LANES)] for j in range(HJ)]
        hvec = [jnp.int32(j * _LANES)
                + lax.iota(jnp.int32, _LANES) for j in range(HJ)]

        def fire_gather(l, b, gsem):
            pltpu.sync_copy(tokt_hbm.at[l, pl.ds(b0, 128)], idx_v.at[b])
            pltpu.async_copy(table_hbm.at[idx_v.at[b]], rows_v.at[b], gsem)

        def wait_gather(b, gsem):
            pltpu.make_async_copy(
                table_hbm.at[idx_v.at[b]], rows_v.at[b], gsem).wait()

        def fire_out(l, b, osem):
            for ht in range(H // 8):
                pltpu.async_copy(
                    outs_v.at[b, pl.ds(ht * 8, 8), pl.ds(0, 128)],
                    out_hbm.at[l, ht, wid],
                    osem,
                )

        def wait_out(b, osem):
            for ht in range(H // 8):
                pltpu.make_async_copy(
                    outs_v.at[b, pl.ds(ht * 8, 8), pl.ds(0, 128)],
                    out_hbm.at[0, ht, wid],
                    osem,
                ).wait()

        def compute(l, b):
            p = [pos_v[l, pl.ds(j * _LANES, _LANES)] for j in range(HJ)]
            KU = 2
            magic = jnp.full((_LANES,), 0x5F3759DF, dtype=jnp.int32)

            @plsc.parallel_loop(0, 128, KU, unroll=2)
            def row_group(i4):
                rr = [i4 + k for k in range(KU)]
                y = [[rows_v[b, r, pl.ds(j * _LANES, _LANES)] + p[j]
                      for j in range(HJ)] for r in rr]
                s = [(yk[0] + yk[1]) + (yk[2] + yk[3]) for yk in y]
                t = [(yk[0] * yk[0] + yk[1] * yk[1])
                     + (yk[2] * yk[2] + yk[3] * yk[3]) for yk in y]
                sr = [lax.rev(sk, (0,)) for sk in s]
                tr = [lax.rev(tk, (0,)) for tk in t]
                cs = [jnp.cumsum(sk) for sk in s]
                csr = [jnp.cumsum(sk) for sk in sr]
                ct = [jnp.cumsum(tk) for tk in t]
                ctr = [jnp.cumsum(tk) for tk in tr]
                tot_s = [c + lax.rev(cr, (0,)) - sk
                         for c, cr, sk in zip(cs, csr, s)]
                tot_t = [c + lax.rev(cr, (0,)) - tk
                         for c, cr, tk in zip(ct, ctr, t)]
                mean = [v * (1.0 / H) for v in tot_s]
                var = [v * (1.0 / H) - m * m
                       for v, m in zip(tot_t, mean)]
                a = [v + _EPS for v in var]
                rs = [plsc.bitcast(
                    magic - lax.shift_right_arithmetic(
                        plsc.bitcast(ak, jnp.int32), 1),
                    jnp.float32) for ak in a]
                rs = [rk * (1.5 - 0.5 * ak * rk * rk)
                      for rk, ak in zip(rs, a)]
                rvec = [jnp.full((_LANES,), r, dtype=jnp.int32) for r in rr]
                for k in range(KU):
                    for j in range(HJ):
                        plsc.store_scatter(
                            outs_v.at[b],
                            [hvec[j], rvec[k]],
                            (y[k][j] - mean[k]) * (rs[k] * g[j]) + bta[j],
                        )

        fire_gather(0, 0, g0)

        def body2(l2, carry):
            l0 = 2 * l2
            l1 = l0 + 1
            fire_gather(l1, 1, g1)
            wait_gather(0, g0)

            @pl.when(l2 > 0)
            def _():
                wait_out(0, o0)

            compute(l0, 0)
            fire_out(l0, 0, o0)

            @pl.when(l2 < L // 2 - 1)
            def _():
                fire_gather(l0 + 2, 0, g0)

            wait_gather(1, g1)

            @pl.when(l2 > 0)
            def _():
                wait_out(1, o1)

            compute(l1, 1)
            fire_out(l1, 1, o1)
            return carry

        lax.fori_loop(0, L // 2, body2, 0)
        wait_out(0, o0)
        wait_out(1, o1)

    return body


def kernel(tokens, start_pos, token_table, pos_table, gamma, beta):
    B, L = tokens.shape
    V, H = token_table.shape
    tokt = tokens.T.astype(jnp.int32)
    pos_slice = lax.dynamic_slice_in_dim(pos_table, start_pos, L, axis=0)
    body = _build(B, L, V, H)
    out5 = body(tokt, token_table, pos_slice, gamma, beta)
    return out5.transpose(2, 4, 0, 1, 3).reshape(B, L, H)


# speedup vs baseline: 1.3132x; 1.0649x over previous
import functools

import jax
import jax.numpy as jnp
from jax import lax
from jax.experimental import pallas as pl
from jax.experimental.pallas import tpu as pltpu
from jax.experimental.pallas import tpu_sc as plsc

_LANES = 16
_EPS = 1e-5


@functools.lru_cache(maxsize=None)
def _build(B, L, V, H):
    info = plsc.get_sparse_core_info()
    NC, NS = info.num_cores, info.num_subcores
    NW = NC * NS
    assert B % (NW * 128) == 0 and H == 4 * _LANES and L % 2 == 0
    HJ = H // _LANES
    CW = 129

    mesh = plsc.VectorSubcoreMesh(core_axis_name="c", subcore_axis_name="s")

    @functools.partial(
        pl.kernel,
        mesh=mesh,
        out_type=jax.ShapeDtypeStruct((L, H // 8, 32, 8, 128), jnp.float32),
        compiler_params=pltpu.CompilerParams(
            needs_layout_passes=False, use_tc_tiling_on_sc=False),
        scratch_types=[
            pltpu.VMEM((2, 128), jnp.int32),
            pltpu.VMEM((2, 128, H), jnp.float32),
            pltpu.VMEM((2, H, CW), jnp.float32),
            pltpu.VMEM((L, H), jnp.float32),
            pltpu.VMEM((H,), jnp.float32),
            pltpu.VMEM((H,), jnp.float32),
            pltpu.SemaphoreType.DMA,
            pltpu.SemaphoreType.DMA,
            pltpu.SemaphoreType.DMA,
            pltpu.SemaphoreType.DMA,
            pltpu.SemaphoreType.DMA,
            pltpu.SemaphoreType.DMA,
        ],
    )
    def body(tokt_hbm, table_hbm, pos_hbm, gamma_hbm, beta_hbm, out_hbm,
             idx_v, rows_v, outs_v, pos_v, gamma_v, beta_v,
             g0, g1, o0, o1, ix0, ix1):
        wid = lax.axis_index("s") * NC + lax.axis_index("c")
        b0 = wid * 128
        pltpu.sync_copy(pos_hbm, pos_v)
        pltpu.sync_copy(gamma_hbm, gamma_v)
        pltpu.sync_copy(beta_hbm, beta_v)
        g = [gamma_v[pl.ds(j * _LANES, _LANES)] for j in range(HJ)]
        bta = [beta_v[pl.ds(j * _LANES, _LANES)] for j in range(HJ)]
        hvec = [jnp.int32(j * _LANES)
                + lax.iota(jnp.int32, _LANES) for j in range(HJ)]

        def fire_idx(l, b, isem):
            pltpu.async_copy(tokt_hbm.at[l, pl.ds(b0, 128)], idx_v.at[b],
                             isem)

        def wait_idx(b, isem):
            pltpu.make_async_copy(
                tokt_hbm.at[0, pl.ds(b0, 128)], idx_v.at[b], isem).wait()

        def fire_gather(b, gsem):
            pltpu.async_copy(table_hbm.at[idx_v.at[b]], rows_v.at[b], gsem)

        def wait_gather(b, gsem):
            pltpu.make_async_copy(
                table_hbm.at[idx_v.at[b]], rows_v.at[b], gsem).wait()

        def fire_out(l, b, osem):
            for ht in range(H // 8):
                pltpu.async_copy(
                    outs_v.at[b, pl.ds(ht * 8, 8), pl.ds(0, 128)],
                    out_hbm.at[l, ht, wid],
                    osem,
                )

        def wait_out(b, osem):
            for ht in range(H // 8):
                pltpu.make_async_copy(
                    outs_v.at[b, pl.ds(ht * 8, 8), pl.ds(0, 128)],
                    out_hbm.at[0, ht, wid],
                    osem,
                ).wait()

        def compute(l, b):
            p = [pos_v[l, pl.ds(j * _LANES, _LANES)] for j in range(HJ)]
            KU = 2
            magic = jnp.full((_LANES,), 0x5F3759DF, dtype=jnp.int32)

            @plsc.parallel_loop(0, 128, KU, unroll=2)
            def row_group(i4):
                rr = [i4 + k for k in range(KU)]
                y = [[rows_v[b, r, pl.ds(j * _LANES, _LANES)] + p[j]
                      for j in range(HJ)] for r in rr]
                s = [(yk[0] + yk[1]) + (yk[2] + yk[3]) for yk in y]
                t = [(yk[0] * yk[0] + yk[1] * yk[1])
                     + (yk[2] * yk[2] + yk[3] * yk[3]) for yk in y]
                sr = [lax.rev(sk, (0,)) for sk in s]
                tr = [lax.rev(tk, (0,)) for tk in t]
                cs = [jnp.cumsum(sk) for sk in s]
                csr = [jnp.cumsum(sk) for sk in sr]
                ct = [jnp.cumsum(tk) for tk in t]
                ctr = [jnp.cumsum(tk) for tk in tr]
                tot_s = [c + lax.rev(cr, (0,)) - sk
                         for c, cr, sk in zip(cs, csr, s)]
                tot_t = [c + lax.rev(cr, (0,)) - tk
                         for c, cr, tk in zip(ct, ctr, t)]
                mean = [v * (1.0 / H) for v in tot_s]
                var = [v * (1.0 / H) - m * m
                       for v, m in zip(tot_t, mean)]
                a = [v + _EPS for v in var]
                rs = [plsc.bitcast(
                    magic - lax.shift_right_arithmetic(
                        plsc.bitcast(ak, jnp.int32), 1),
                    jnp.float32) for ak in a]
                rs = [rk * (1.5 - 0.5 * ak * rk * rk)
                      for rk, ak in zip(rs, a)]
                rvec = [jnp.full((_LANES,), r, dtype=jnp.int32) for r in rr]
                for k in range(KU):
                    for j in range(HJ):
                        plsc.store_scatter(
                            outs_v.at[b],
                            [hvec[j], rvec[k]],
                            (y[k][j] - mean[k]) * (rs[k] * g[j]) + bta[j],
                        )

        pltpu.sync_copy(tokt_hbm.at[0, pl.ds(b0, 128)], idx_v.at[0])
        fire_gather(0, g0)
        fire_idx(1, 1, ix1)

        def body2(l2, carry):
            l0 = 2 * l2
            l1 = l0 + 1
            wait_idx(1, ix1)
            fire_gather(1, g1)
            wait_gather(0, g0)

            @pl.when(l2 < L // 2 - 1)
            def _():
                fire_idx(l0 + 2, 0, ix0)

            @pl.when(l2 > 0)
            def _():
                wait_out(0, o0)

            compute(l0, 0)
            fire_out(l0, 0, o0)

            @pl.when(l2 < L // 2 - 1)
            def _():
                wait_idx(0, ix0)
                fire_gather(0, g0)

            wait_gather(1, g1)

            @pl.when(l2 < L // 2 - 1)
            def _():
                fire_idx(l1 + 2, 1, ix1)

            @pl.when(l2 > 0)
            def _():
                wait_out(1, o1)

            compute(l1, 1)
            fire_out(l1, 1, o1)
            return carry

        lax.fori_loop(0, L // 2, body2, 0)
        wait_out(0, o0)
        wait_out(1, o1)

    return body


def kernel(tokens, start_pos, token_table, pos_table, gamma, beta):
    B, L = tokens.shape
    V, H = token_table.shape
    tokt = tokens.T.astype(jnp.int32)
    pos_slice = lax.dynamic_slice_in_dim(pos_table, start_pos, L, axis=0)
    body = _build(B, L, V, H)
    out5 = body(tokt, token_table, pos_slice, gamma, beta)
    return out5.transpose(2, 4, 0, 1, 3).reshape(B, L, H)
